# baseline (speedup 1.0000x reference)
import functools

import jax
import jax.numpy as jnp
from jax import lax
from jax.experimental import pallas as pl
from jax.experimental.pallas import tpu as pltpu
from jax.experimental.pallas import tpu_sc as plsc

_DIM = 64
_SEGW = 2048
_ROWW = 128
_FLUSH = 128
_SEG_PER_W = 31
_PAD = 128
_SENT = 1 << 29


def _extract_side(tbl_h, key_h, pos_h, rows_h,
                  key_v, pos_v, buf_a, buf_b, stage_v, spos_v,
                  sem_a, sem_b, sem_f, w, n_idx, n_rows):
    trash = n_idx
    nseg = -(-n_rows // _SEGW)
    wmax = (-(-n_rows // 128)) * 128 - _SEGW
    s0 = w * _SEG_PER_W
    ns = jnp.minimum(_SEG_PER_W, jnp.maximum(nseg - s0, 0))
    n_pad = n_idx + _PAD
    pltpu.sync_copy(key_h, key_v.at[pl.ds(0, n_pad)])
    pltpu.sync_copy(pos_h, pos_v.at[pl.ds(0, n_pad)])

    iota = lax.iota(jnp.int32, 16)

    def init_spos():
        zero = jnp.zeros((16,), jnp.int32)
        tr16 = jnp.full((16,), trash, jnp.int32)
        for q in range(_FLUSH // 16):
            plsc.store_scatter(spos_v, [zero, q * 16 + iota], tr16)

    def flush():
        pltpu.async_copy(stage_v, rows_h.at[spos_v.at[0]], sem_f).wait()
        init_spos()

    init_spos()

    target = s0 * _SEGW

    def bs_body(_, lo_hi):
        lo, hi = lo_hi
        mid = (lo + hi) // 2
        v = key_v[pl.ds(mid, 16)][0]
        new_lo = jnp.where(v < target, mid + 1, lo)
        new_hi = jnp.where(v < target, hi, mid)
        return (new_lo, new_hi)

    p0, _ = lax.fori_loop(0, 16, bs_body, (jnp.int32(0), jnp.int32(n_idx)))

    def seg_body(seg_rel, carry):
        p, fill = carry
        seg = s0 + seg_rel
        c0 = pl.multiple_of(jnp.minimum(seg * _SEGW, wmax), 128)
        bound = (seg + 1) * _SEGW

        def pc_cond(st):
            return st[1]

        def pc_body(st):
            p2, _ = st
            jc = jnp.minimum(p2 + iota, n_pad - 1)
            keys = plsc.load_gather(key_v, [jc])
            cnt = plsc.all_reduce_population_count(keys < bound)[0]
            return (p2 + cnt, cnt == 16)

        p_end, _ = lax.while_loop(pc_cond, pc_body, (p, True))
        cnt_seg = p_end - p
        n_chunks = lax.div(cnt_seg + _FLUSH - 1, _FLUSH)

        def src(tr):
            return tbl_h.at[pl.ds(tr * 8, 8), pl.ds(c0, _SEGW)]

        def fire(tr):
            if tr % 2 == 0:
                pltpu.async_copy(src(tr), buf_a, sem_a)
            else:
                pltpu.async_copy(src(tr), buf_b, sem_b)

        def wait(tr):
            if tr % 2 == 0:
                pltpu.make_async_copy(src(tr), buf_a, sem_a).wait()
            else:
                pltpu.make_async_copy(src(tr), buf_b, sem_b).wait()

        def chunk_body(ck, fill2):
            ck_p = p + ck * _FLUSH
            ck_cnt = jnp.minimum(cnt_seg - ck * _FLUSH, _FLUSH)
            ngr = lax.div(ck_cnt + 15, 16)
            rounded = ngr * 16

            @pl.when(fill2 + rounded > _FLUSH)
            def _():
                flush()

            fill3 = jnp.where(fill2 + rounded > _FLUSH, 0, fill2)

            fire(0)
            for tr in range(8):
                if tr < 7:
                    fire(tr + 1)
                wait(tr)
                buf = buf_a if tr % 2 == 0 else buf_b

                def pass_body(g, carry2, tr=tr, buf=buf):
                    j16 = ck_p + g * 16 + iota
                    jc = jnp.minimum(j16, n_pad - 1)
                    keys = plsc.load_gather(key_v, [jc])
                    lanes = jnp.clip(keys - c0, 0, _SEGW - 1)
                    slots = fill3 + g * 16 + iota
                    for dl in range(8):
                        dv = jnp.full((16,), dl, jnp.int32)
                        vals = plsc.load_gather(buf, [dv, lanes])
                        plsc.store_scatter(stage_v, [slots,
                                                     jnp.full((16,), tr * 8 + dl,
                                                              jnp.int32)], vals)
                    if tr == 7:
                        m = (g * 16 + iota) < ck_cnt
                        positions = plsc.load_gather(pos_v, [jc])
                        positions = jnp.where(m, positions,
                                              jnp.full((16,), trash, jnp.int32))
                        plsc.store_scatter(
                            spos_v, [jnp.zeros((16,), jnp.int32), slots],
                            positions)
                    return carry2

                lax.fori_loop(0, ngr, pass_body, 0)
            return fill3 + ck_cnt

        fill_new = lax.fori_loop(0, n_chunks, chunk_body, fill)
        return (p_end, fill_new)

    lax.fori_loop(0, ns, seg_body, (p0, jnp.int32(0)))
    flush()


def _build_extract(n_ent, n_rel, n_eidx, n_ridx):
    mesh = plsc.VectorSubcoreMesh(core_axis_name="c", subcore_axis_name="s")

    @functools.partial(
        pl.kernel,
        mesh=mesh,
        out_type=(jax.ShapeDtypeStruct((n_eidx + 8, _ROWW), jnp.float32),
                  jax.ShapeDtypeStruct((n_ridx + 8, _ROWW), jnp.float32)),
        compiler_params=pltpu.CompilerParams(needs_layout_passes=False),
        scratch_types=[
            pltpu.VMEM((n_eidx + _PAD,), jnp.int32),
            pltpu.VMEM((n_eidx + _PAD,), jnp.int32),
            pltpu.VMEM((8, _SEGW), jnp.float32),
            pltpu.VMEM((8, _SEGW), jnp.float32),
            pltpu.VMEM((_FLUSH, _ROWW), jnp.float32),
            pltpu.VMEM((1, _FLUSH), jnp.int32),
            pltpu.SemaphoreType.DMA,
            pltpu.SemaphoreType.DMA,
            pltpu.SemaphoreType.DMA,
        ],
    )
    def extract(entT_h, relT_h, ekey_h, epos_h, rkey_h, rpos_h,
                erows_h, rrows_h,
                key_v, pos_v, buf_a, buf_b, stage_v, spos_v,
                sem_a, sem_b, sem_f):
        cid = lax.axis_index("c")
        sid = lax.axis_index("s")

        @pl.when(cid == 0)
        def _():
            _extract_side(entT_h, ekey_h, epos_h, erows_h,
                          key_v, pos_v, buf_a, buf_b, stage_v, spos_v,
                          sem_a, sem_b, sem_f, sid, n_eidx, n_ent)

        @pl.when(cid == 1)
        def _():
            _extract_side(relT_h, rkey_h, rpos_h, rrows_h,
                          key_v, pos_v, buf_a, buf_b, stage_v, spos_v,
                          sem_a, sem_b, sem_f, sid, n_ridx, n_rel)

    return extract


def _build_combine(batch):
    info = plsc.get_sparse_core_info()
    nc, ns_sub = info.num_cores, info.num_subcores
    nw = nc * ns_sub
    b_per_w = batch // nw
    chunk = 256
    n_chunks = b_per_w // chunk
    n_groups = chunk // 16

    mesh = plsc.VectorSubcoreMesh(core_axis_name="c", subcore_axis_name="s")

    @functools.partial(
        pl.kernel,
        mesh=mesh,
        out_type=jax.ShapeDtypeStruct((batch,), jnp.float32),
        compiler_params=pltpu.CompilerParams(needs_layout_passes=False),
        scratch_types=[
            pltpu.VMEM((chunk, _ROWW), jnp.float32),
            pltpu.VMEM((chunk, _ROWW), jnp.float32),
            pltpu.VMEM((chunk, _ROWW), jnp.float32),
            pltpu.VMEM((b_per_w,), jnp.float32),
            pltpu.VMEM((128,), jnp.float32),
            pltpu.VMEM((128,), jnp.float32),
            pltpu.SemaphoreType.DMA,
        ],
    )
    def combine(erows_h, rrows_h, w_h, b_h, out_h,
                h_v, t_v, r_v, out_v, w_v, b_v, sem):
        wid = lax.axis_index("s") * nc + lax.axis_index("c")
        base = wid * b_per_w
        pltpu.sync_copy(w_h, w_v)
        pltpu.sync_copy(b_h, b_v)
        wv = w_v[pl.ds(0, 16)]
        bv = b_v[pl.ds(0, 16)]
        iota = lax.iota(jnp.int32, 16)

        def chunk_body(c, carry):
            cb = base + c * chunk
            cp_h = pltpu.async_copy(erows_h.at[pl.ds(cb, chunk)], h_v, sem)
            cp_t = pltpu.async_copy(
                erows_h.at[pl.ds(batch + cb, chunk)], t_v, sem)
            cp_r = pltpu.async_copy(rrows_h.at[pl.ds(cb, chunk)], r_v, sem)
            cp_h.wait()
            cp_t.wait()
            cp_r.wait()

            def group_body(g, carry2):
                rows = g * 16 + iota
                acc = jnp.zeros((16,), jnp.float32)
                for d in range(_DIM):
                    dv = jnp.full((16,), d, jnp.int32)
                    acc = acc + (plsc.load_gather(h_v, [rows, dv])
                                 * plsc.load_gather(r_v, [rows, dv])
                                 * plsc.load_gather(t_v, [rows, dv]))
                z = acc * wv + bv
                out_v[pl.ds(c * chunk + g * 16, 16)] = (
                    1.0 / (1.0 + jnp.exp(-z)))
                return carry2

            lax.fori_loop(0, n_groups, group_body, 0)
            return carry

        lax.fori_loop(0, n_chunks, chunk_body, 0)
        pltpu.sync_copy(out_v, out_h.at[pl.ds(base, b_per_w)])

    return combine


def _side_prep(idx):
    order = jnp.argsort(idx).astype(jnp.int32)
    si = jnp.take(idx, order).astype(jnp.int32)
    key = jnp.concatenate([si, jnp.full((_PAD,), _SENT, jnp.int32)])
    pos = jnp.concatenate([order, jnp.zeros((_PAD,), jnp.int32)])
    return key, pos


def kernel(x, entity_table, rel_table, lin_w, lin_b):
    batch = x.shape[0]
    n_ent = entity_table.shape[0]
    n_rel = rel_table.shape[0]
    xi = x.astype(jnp.int32)
    eidx = jnp.concatenate([xi[:, 0], xi[:, 2]])
    ridx = xi[:, 1]
    ekey, epos = _side_prep(eidx)
    rkey, rpos = _side_prep(ridx)
    wvec = jnp.full((128,), lin_w[0, 0], jnp.float32)
    bvec = jnp.full((128,), lin_b[0], jnp.float32)

    extract = _build_extract(n_ent, n_rel, eidx.shape[0], ridx.shape[0])
    erows, rrows = extract(entity_table.T, rel_table.T,
                           ekey, epos, rkey, rpos)
    combine = _build_combine(batch)
    return combine(erows, rrows, wvec, bvec)


# speedup vs baseline: 2.5265x; 2.5265x over previous
import functools

import jax
import jax.numpy as jnp
from jax import lax
from jax.experimental import pallas as pl
from jax.experimental.pallas import tpu as pltpu
from jax.experimental.pallas import tpu_sc as plsc

_DIM = 64
_STRIPW = 256
_SHIFT = 8
_ROWW = 128
_FLUSH = 128
_SPW = 245
_NBUF = 2
_PAD = 128
_SENT = 1 << 29


def _extract_side(tbl_h, key_h, pos_h, rows_h,
                  key_v, pos_v, buf_v, stage_v, spos_v, sems,
                  w, n_idx, n_strips, n_rows):
    trash = n_idx
    wmax = (-(-n_rows // 128)) * 128 - _STRIPW
    s0 = w * _SPW
    ns = jnp.minimum(_SPW, jnp.maximum(n_strips - s0, 0))
    n_pad = n_idx + _PAD
    pltpu.sync_copy(key_h, key_v.at[pl.ds(0, n_pad)])
    pltpu.sync_copy(pos_h, pos_v.at[pl.ds(0, n_pad)])

    def strip_src(k):
        col = pl.multiple_of(
            jnp.minimum((s0 + k) * _STRIPW, wmax), 128)
        return tbl_h.at[:, pl.ds(col, _STRIPW)]

    def fire(k):
        slot = lax.rem(k, _NBUF)
        pltpu.async_copy(strip_src(k), buf_v.at[slot], sems.at[slot])

    def wait(k):
        slot = lax.rem(k, _NBUF)
        pltpu.make_async_copy(strip_src(k), buf_v.at[slot],
                              sems.at[slot]).wait()

    iota = lax.iota(jnp.int32, 16)

    def init_spos():
        zero = jnp.zeros((16,), jnp.int32)
        tr = jnp.full((16,), trash, jnp.int32)
        for q in range(_FLUSH // 16):
            plsc.store_scatter(spos_v, [zero, q * 16 + iota], tr)

    def flush():
        pltpu.async_copy(stage_v, rows_h.at[spos_v.at[0]], sems.at[_NBUF]).wait()
        init_spos()

    init_spos()

    lo0 = jnp.int32(0)
    target = s0 * _STRIPW

    def bs_body(_, lo_hi):
        lo, hi = lo_hi
        mid = (lo + hi) // 2
        v = key_v[pl.ds(mid, 16)][0]
        new_lo = jnp.where(v < target, mid + 1, lo)
        new_hi = jnp.where(v < target, hi, mid)
        return (new_lo, new_hi)

    p0, _ = lax.fori_loop(0, 16, bs_body, (lo0, jnp.int32(n_idx)))

    @pl.when(ns > 0)
    def _():
        for k0 in range(_NBUF - 1):
            @pl.when(k0 < ns)
            def _():
                fire(k0)

        def strip_body(k, carry):
            p, fill = carry

            @pl.when(k + _NBUF - 1 < ns)
            def _():
                fire(k + _NBUF - 1)

            wait(k)
            bound = (s0 + k + 1) * _STRIPW
            par = lax.rem(k, _NBUF)
            parv = jnp.full((16,), par, jnp.int32)

            def group_cond(state):
                _, _, go = state
                return go

            def group_body(state):
                p2, fill2, _ = state
                jc = jnp.minimum(p2 + iota, n_pad - 1)
                keys = plsc.load_gather(key_v, [jc])
                m = keys < bound
                cnt = plsc.all_reduce_population_count(m)[0]
                positions = plsc.load_gather(pos_v, [jc])
                positions = jnp.where(m, positions,
                                      jnp.full((16,), trash, jnp.int32))
                base_l = jnp.minimum(
                    (keys >> _SHIFT) << _SHIFT, wmax)
                lanes = jnp.clip(keys - base_l, 0, _STRIPW - 1)

                @pl.when(fill2 + 16 > _FLUSH)
                def _():
                    flush()

                fill3 = jnp.where(fill2 + 16 > _FLUSH, 0, fill2)

                @pl.when(cnt > 0)
                def _():
                    slots = fill3 + iota
                    for d in range(_DIM):
                        dv = jnp.full((16,), d, jnp.int32)
                        vals = plsc.load_gather(buf_v, [parv, dv, lanes])
                        plsc.store_scatter(stage_v, [slots, dv], vals)
                    plsc.store_scatter(spos_v,
                                       [jnp.zeros((16,), jnp.int32), slots],
                                       positions)

                return (p2 + cnt, fill3 + cnt, cnt == 16)

            p_new, fill_new, _ = lax.while_loop(
                group_cond, group_body, (p, fill, True))
            return (p_new, fill_new)

        lax.fori_loop(0, ns, strip_body, (p0, jnp.int32(0)))
        flush()


def _build_extract(n_ent, n_rel, n_eidx, n_ridx):
    e_strips = -(-n_ent // _STRIPW)
    r_strips = -(-n_rel // _STRIPW)

    mesh = plsc.VectorSubcoreMesh(core_axis_name="c", subcore_axis_name="s")

    @functools.partial(
        pl.kernel,
        mesh=mesh,
        out_type=(jax.ShapeDtypeStruct((n_eidx + 8, _ROWW), jnp.float32),
                  jax.ShapeDtypeStruct((n_ridx + 8, _ROWW), jnp.float32)),
        compiler_params=pltpu.CompilerParams(needs_layout_passes=False),
        scratch_types=[
            pltpu.VMEM((n_eidx + _PAD,), jnp.int32),
            pltpu.VMEM((n_eidx + _PAD,), jnp.int32),
            pltpu.VMEM((_NBUF, _DIM, _STRIPW), jnp.float32),
            pltpu.VMEM((_FLUSH, _ROWW), jnp.float32),
            pltpu.VMEM((1, _FLUSH), jnp.int32),
            pltpu.SemaphoreType.DMA((_NBUF + 1,)),
        ],
    )
    def extract(entT_h, relT_h, ekey_h, epos_h, rkey_h, rpos_h,
                erows_h, rrows_h,
                key_v, pos_v, buf_v, stage_v, spos_v, sems):
        cid = lax.axis_index("c")
        sid = lax.axis_index("s")

        @pl.when(cid == 0)
        def _():
            _extract_side(entT_h, ekey_h, epos_h, erows_h,
                          key_v, pos_v, buf_v, stage_v, spos_v, sems,
                          sid, n_eidx, e_strips, n_ent)

        @pl.when(cid == 1)
        def _():
            _extract_side(relT_h, rkey_h, rpos_h, rrows_h,
                          key_v, pos_v, buf_v, stage_v, spos_v, sems,
                          sid, n_ridx, r_strips, n_rel)

    return extract


def _build_combine(batch):
    info = plsc.get_sparse_core_info()
    nc, ns_sub = info.num_cores, info.num_subcores
    nw = nc * ns_sub
    b_per_w = batch // nw
    chunk = 256
    n_chunks = b_per_w // chunk
    n_groups = chunk // 16

    mesh = plsc.VectorSubcoreMesh(core_axis_name="c", subcore_axis_name="s")

    @functools.partial(
        pl.kernel,
        mesh=mesh,
        out_type=jax.ShapeDtypeStruct((batch,), jnp.float32),
        compiler_params=pltpu.CompilerParams(needs_layout_passes=False),
        scratch_types=[
            pltpu.VMEM((chunk, _ROWW), jnp.float32),
            pltpu.VMEM((chunk, _ROWW), jnp.float32),
            pltpu.VMEM((chunk, _ROWW), jnp.float32),
            pltpu.VMEM((b_per_w,), jnp.float32),
            pltpu.VMEM((128,), jnp.float32),
            pltpu.VMEM((128,), jnp.float32),
            pltpu.SemaphoreType.DMA,
        ],
    )
    def combine(erows_h, rrows_h, w_h, b_h, out_h,
                h_v, t_v, r_v, out_v, w_v, b_v, sem):
        wid = lax.axis_index("s") * nc + lax.axis_index("c")
        base = wid * b_per_w
        pltpu.sync_copy(w_h, w_v)
        pltpu.sync_copy(b_h, b_v)
        wv = w_v[pl.ds(0, 16)]
        bv = b_v[pl.ds(0, 16)]
        iota = lax.iota(jnp.int32, 16)

        def chunk_body(c, carry):
            cb = base + c * chunk
            cp_h = pltpu.async_copy(erows_h.at[pl.ds(cb, chunk)], h_v, sem)
            cp_t = pltpu.async_copy(
                erows_h.at[pl.ds(batch + cb, chunk)], t_v, sem)
            cp_r = pltpu.async_copy(rrows_h.at[pl.ds(cb, chunk)], r_v, sem)
            cp_h.wait()
            cp_t.wait()
            cp_r.wait()

            def group_body(g, carry2):
                rows = g * 16 + iota
                acc = jnp.zeros((16,), jnp.float32)
                for d in range(_DIM):
                    dv = jnp.full((16,), d, jnp.int32)
                    acc = acc + (plsc.load_gather(h_v, [rows, dv])
                                 * plsc.load_gather(r_v, [rows, dv])
                                 * plsc.load_gather(t_v, [rows, dv]))
                z = acc * wv + bv
                out_v[pl.ds(c * chunk + g * 16, 16)] = (
                    1.0 / (1.0 + jnp.exp(-z)))
                return carry2

            lax.fori_loop(0, n_groups, group_body, 0)
            return carry

        lax.fori_loop(0, n_chunks, chunk_body, 0)
        pltpu.sync_copy(out_v, out_h.at[pl.ds(base, b_per_w)])

    return combine


def _side_prep(idx):
    order = jnp.argsort(idx).astype(jnp.int32)
    si = jnp.take(idx, order).astype(jnp.int32)
    key = jnp.concatenate([si, jnp.full((_PAD,), _SENT, jnp.int32)])
    pos = jnp.concatenate([order, jnp.zeros((_PAD,), jnp.int32)])
    return key, pos


def kernel(x, entity_table, rel_table, lin_w, lin_b):
    batch = x.shape[0]
    n_ent = entity_table.shape[0]
    n_rel = rel_table.shape[0]
    xi = x.astype(jnp.int32)
    eidx = jnp.concatenate([xi[:, 0], xi[:, 2]])
    ridx = xi[:, 1]
    ekey, epos = _side_prep(eidx)
    rkey, rpos = _side_prep(ridx)
    wvec = jnp.full((128,), lin_w[0, 0], jnp.float32)
    bvec = jnp.full((128,), lin_b[0], jnp.float32)

    extract = _build_extract(n_ent, n_rel, eidx.shape[0], ridx.shape[0])
    erows, rrows = extract(entity_table.T, rel_table.T,
                           ekey, epos, rkey, rpos)
    combine = _build_combine(batch)
    return combine(erows, rrows, wvec, bvec)
